# Initial kernel scaffold; baseline (speedup 1.0000x reference)
import jax
import jax.numpy as jnp
from jax.experimental import pallas as pl


def kernel(token_ids, emb):
    raise NotImplementedError("write your pallas kernel here")


# speedup vs baseline: 4.0862x; 4.0862x over previous
import functools

import jax
import jax.numpy as jnp
from jax import lax
from jax.experimental import pallas as pl
from jax.experimental.pallas import tpu as pltpu
from jax.experimental.pallas import tpu_sc as plsc

NUM_EMB = 100000
D = 64
B_TOTAL = 4096 * 50
C = 128


@functools.cache
def _build_lookup():
    info = plsc.get_sparse_core_info()
    nc, ns = info.num_cores, info.num_subcores
    nw = nc * ns
    b_per_w = B_TOTAL // nw
    nchunk = b_per_w // C

    mesh = plsc.VectorSubcoreMesh(core_axis_name="c", subcore_axis_name="s")

    def body(idx_hbm, table_hbm, out_hbm, idx_v, rows_v, gsem):
        wid = lax.axis_index("s") * nc + lax.axis_index("c")
        base = wid * b_per_w
        pltpu.sync_copy(idx_hbm.at[wid], idx_v)

        @pl.loop(0, nchunk)
        def _(j):
            pltpu.async_copy(table_hbm.at[idx_v.at[j]], rows_v, gsem).wait()
            pltpu.sync_copy(rows_v, out_hbm.at[pl.ds(base + j * C, C)])

    return pl.kernel(
        body,
        out_type=jax.ShapeDtypeStruct((B_TOTAL, D), jnp.float32),
        mesh=mesh,
        scratch_types=[
            pltpu.VMEM((nchunk, C), jnp.int32),
            pltpu.VMEM((C, D), jnp.float32),
            pltpu.SemaphoreType.DMA,
        ],
        compiler_params=pltpu.CompilerParams(use_tc_tiling_on_sc=False),
    ), nw, nchunk


def kernel(token_ids, emb):
    lookup, nw, nchunk = _build_lookup()
    ids = token_ids.reshape(nw, nchunk, C)
    out = lookup(ids, emb)
    return out.reshape(*token_ids.shape, D)


# speedup vs baseline: 4.6858x; 1.1467x over previous
import functools

import jax
import jax.numpy as jnp
from jax import lax
from jax.experimental import pallas as pl
from jax.experimental.pallas import tpu as pltpu
from jax.experimental.pallas import tpu_sc as plsc

NUM_EMB = 100000
D = 64
B_TOTAL = 4096 * 50
C = 128


@functools.cache
def _build_lookup():
    info = plsc.get_sparse_core_info()
    nc, ns = info.num_cores, info.num_subcores
    nw = nc * ns
    b_per_w = B_TOTAL // nw
    nchunk = b_per_w // C

    mesh = plsc.VectorSubcoreMesh(core_axis_name="c", subcore_axis_name="s")

    nbuf = 8
    lead = 4

    def body(idx_hbm, table_hbm, out_hbm, idx_v, rows_v, gsem, ssem):
        wid = lax.axis_index("s") * nc + lax.axis_index("c")
        base = wid * b_per_w
        pltpu.sync_copy(idx_hbm.at[wid], idx_v)

        def gather(j, b):
            pltpu.async_copy(table_hbm.at[idx_v.at[j]], rows_v.at[b], gsem.at[b])

        def gather_wait(j, b):
            pltpu.make_async_copy(
                table_hbm.at[idx_v.at[j]], rows_v.at[b], gsem.at[b]).wait()

        def scatter(j, b):
            pltpu.async_copy(
                rows_v.at[b], out_hbm.at[pl.ds(base + j * C, C)], ssem.at[b])

        def scatter_wait(j, b):
            pltpu.make_async_copy(
                rows_v.at[b], out_hbm.at[pl.ds(base + j * C, C)], ssem.at[b]).wait()

        for p in range(lead):
            gather(p, p)

        @pl.loop(0, nchunk)
        def _(j):
            b = lax.rem(j, nbuf)
            jn = j + lead

            @pl.when(jn < nchunk)
            def _():
                bn = lax.rem(jn, nbuf)

                @pl.when(jn >= nbuf)
                def _():
                    scatter_wait(jn - nbuf, bn)

                gather(jn, bn)

            gather_wait(j, b)
            scatter(j, b)

        for t in range(nbuf):
            j = nchunk - nbuf + t
            scatter_wait(j, j % nbuf)

    return pl.kernel(
        body,
        out_type=jax.ShapeDtypeStruct((B_TOTAL, D), jnp.float32),
        mesh=mesh,
        scratch_types=[
            pltpu.VMEM((nchunk, C), jnp.int32),
            pltpu.VMEM((nbuf, C, D), jnp.float32),
            pltpu.SemaphoreType.DMA((nbuf,)),
            pltpu.SemaphoreType.DMA((nbuf,)),
        ],
        compiler_params=pltpu.CompilerParams(use_tc_tiling_on_sc=False),
    ), nw, nchunk


def kernel(token_ids, emb):
    lookup, nw, nchunk = _build_lookup()
    ids = token_ids.reshape(nw, nchunk, C)
    out = lookup(ids, emb)
    return out.reshape(*token_ids.shape, D)
